# Initial kernel scaffold; baseline (speedup 1.0000x reference)
import jax
import jax.numpy as jnp
from jax.experimental import pallas as pl


def kernel(utteranceTokens, table, W, b):
    raise NotImplementedError("write your pallas kernel here")


# speedup vs baseline: 2.2454x; 2.2454x over previous
import functools

import jax
import jax.numpy as jnp
from jax import lax
from jax.experimental import pallas as pl
from jax.experimental.pallas import tpu as pltpu
from jax.experimental.pallas import tpu_sc as plsc

VOCAB_SIZE = 1000000
EMB_D = 32
N_CLS = 100
BATCH_N = 4096
SEQ_N = 200

NUM_CORES = 2
NUM_SUBCORES = 16
NUM_WORKERS = NUM_CORES * NUM_SUBCORES
B_PER_W = BATCH_N // NUM_WORKERS
N_CHUNK = 2
CHUNK = SEQ_N // N_CHUNK
LANES = 16
D_HALF = EMB_D // LANES


def _sc_pool(idx3, table):
    mesh = plsc.VectorSubcoreMesh(
        core_axis_name="c", subcore_axis_name="s",
        num_cores=NUM_CORES, num_subcores=NUM_SUBCORES)

    @functools.partial(
        pl.kernel,
        out_type=jax.ShapeDtypeStruct((BATCH_N, EMB_D), jnp.float32),
        mesh=mesh,
        compiler_params=pltpu.CompilerParams(use_tc_tiling_on_sc=False),
        scratch_types=[
            pltpu.VMEM((B_PER_W, N_CHUNK, CHUNK), jnp.int32),
            pltpu.VMEM((2, N_CHUNK, CHUNK, EMB_D), jnp.float32),
            pltpu.VMEM((B_PER_W, EMB_D), jnp.float32),
            pltpu.SemaphoreType.DMA,
            pltpu.SemaphoreType.DMA,
        ],
    )
    def pool(idx_hbm, table_hbm, out_hbm, idx_v, rows_v, out_v, sem0, sem1):
        wid = lax.axis_index("s") * NUM_CORES + lax.axis_index("c")
        base = wid * B_PER_W
        pltpu.sync_copy(idx_hbm.at[pl.ds(base, B_PER_W)], idx_v)
        sems = (sem0, sem1)

        def fire(u, buf):
            @pl.when(u < B_PER_W)
            def _():
                for c in range(N_CHUNK):
                    pltpu.async_copy(
                        table_hbm.at[idx_v.at[u, c]], rows_v.at[buf, c],
                        sems[buf])

        def drain(u, buf):
            for c in range(N_CHUNK):
                pltpu.make_async_copy(
                    table_hbm.at[idx_v.at[u, c]], rows_v.at[buf, c],
                    sems[buf]).wait()

        def accum(u, buf):
            def body(s, carry):
                a0, a1 = carry
                for c in range(N_CHUNK):
                    a0 = a0 + rows_v[buf, c, s, pl.ds(0, LANES)]
                    a1 = a1 + rows_v[buf, c, s, pl.ds(LANES, LANES)]
                return a0, a1
            zero = jnp.zeros((LANES,), jnp.float32)
            a0, a1 = lax.fori_loop(0, CHUNK, body, (zero, zero))
            out_v[u, pl.ds(0, LANES)] = a0
            out_v[u, pl.ds(LANES, LANES)] = a1

        fire(0, 0)
        fire(1, 1)

        def outer(i, carry):
            g = 2 * i
            drain(g, 0)
            accum(g, 0)
            fire(g + 2, 0)
            drain(g + 1, 1)
            accum(g + 1, 1)
            fire(g + 3, 1)
            return carry

        lax.fori_loop(0, B_PER_W // 2, outer, 0)
        pltpu.sync_copy(out_v, out_hbm.at[pl.ds(base, B_PER_W)])

    return pool(idx3, table)


def _tc_linear(pooled, wt, b2):
    bm = 512

    def body(x_ref, w_ref, b_ref, o_ref):
        o_ref[...] = jnp.dot(
            x_ref[...], w_ref[...],
            precision=jax.lax.Precision.HIGHEST,
            preferred_element_type=jnp.float32) + b_ref[...]

    return pl.pallas_call(
        body,
        grid=(BATCH_N // bm,),
        in_specs=[
            pl.BlockSpec((bm, EMB_D), lambda i: (i, 0)),
            pl.BlockSpec((EMB_D, N_CLS), lambda i: (0, 0)),
            pl.BlockSpec((1, N_CLS), lambda i: (0, 0)),
        ],
        out_specs=pl.BlockSpec((bm, N_CLS), lambda i: (i, 0)),
        out_shape=jax.ShapeDtypeStruct((BATCH_N, N_CLS), jnp.float32),
    )(pooled, wt, b2)


def kernel(utteranceTokens, table, W, b):
    idx3 = utteranceTokens.astype(jnp.int32).reshape(BATCH_N, N_CHUNK, CHUNK)
    pooled = _sc_pool(idx3, table)
    wt = (W.astype(jnp.float32) * (1.0 / SEQ_N)).T
    b2 = b.reshape(1, N_CLS)
    return _tc_linear(pooled, wt, b2)
